# baseline (speedup 1.0000x reference)
import jax
import jax.numpy as jnp
import numpy as np
from jax.experimental import pallas as pl
from jax.experimental.pallas import tpu as pltpu

_PACK = 8
_BLK = 2000


def _mlp_kernel(x0, x1, x2, x3, x4, x5, x6, x7,
                w1_ref, b1_ref, w2_ref, b2_ref, o_ref):
    c = o_ref.shape[1]
    xs = (x0, x1, x2, x3, x4, x5, x6, x7)
    xp = jnp.concatenate([x[...] for x in xs], axis=1)
    h = jnp.dot(xp, w1_ref[...], preferred_element_type=jnp.float32) + b1_ref[...]
    h = 0.5 * h * (1.0 + jax.lax.erf(h * np.float32(1.0 / np.sqrt(2.0))))
    o = jnp.dot(h, w2_ref[...], preferred_element_type=jnp.float32) + b2_ref[...]
    for s in range(_PACK):
        o_ref[s * _BLK:(s + 1) * _BLK, :] = o[:, c * s:c * (s + 1)]


def kernel(node_features, edge_features, edge_index, node_tiers,
           Wq1, Wk1, Wv1, We1, Wo1, Wq2, Wk2, Wv2, We2, Wo2,
           Wc1, bc1, Wc2, bc2):
    E, DE = edge_features.shape
    C = Wc2.shape[1]
    din = _PACK * DE
    dout = _PACK * C
    nblk = E // (_PACK * _BLK)

    eye = jnp.eye(_PACK, dtype=jnp.float32)
    w1 = jnp.kron(eye, Wc1.astype(jnp.float32))
    w2 = jnp.kron(eye, Wc2.astype(jnp.float32))
    b1 = jnp.tile(bc1.astype(jnp.float32), _PACK)[None, :]
    b2 = jnp.tile(bc2.astype(jnp.float32), _PACK)[None, :]

    in_specs = [
        pl.BlockSpec((_BLK, DE), lambda i, s=s: (_PACK * i + s, 0))
        for s in range(_PACK)
    ] + [
        pl.BlockSpec((din, din), lambda i: (0, 0)),
        pl.BlockSpec((1, din), lambda i: (0, 0)),
        pl.BlockSpec((din, dout), lambda i: (0, 0)),
        pl.BlockSpec((1, dout), lambda i: (0, 0)),
    ]

    out = pl.pallas_call(
        _mlp_kernel,
        grid=(nblk,),
        in_specs=in_specs,
        out_specs=pl.BlockSpec((_PACK * _BLK, C), lambda i: (i, 0)),
        out_shape=jax.ShapeDtypeStruct((E, C), jnp.float32),
        compiler_params=pltpu.CompilerParams(
            dimension_semantics=("arbitrary",),
        ),
    )(*([edge_features] * _PACK), w1, b1, w2, b2)
    return out


# speedup vs baseline: 1.0027x; 1.0027x over previous
import jax
import jax.numpy as jnp
import numpy as np
from jax.experimental import pallas as pl
from jax.experimental.pallas import tpu as pltpu

_PACK = 8
_BLK = 2000


def _mlp_kernel(x_ref, w1_ref, b1_ref, w2_ref, b2_ref, o_ref):
    c = o_ref.shape[1]
    xp = jnp.concatenate(
        [x_ref[s * _BLK:(s + 1) * _BLK, :] for s in range(_PACK)],
        axis=1)
    h = jnp.dot(xp, w1_ref[...], preferred_element_type=jnp.float32) + b1_ref[...]
    h = 0.5 * h * (1.0 + jax.lax.erf(h * np.float32(1.0 / np.sqrt(2.0))))
    o = jnp.dot(h, w2_ref[...], preferred_element_type=jnp.float32) + b2_ref[...]
    for s in range(_PACK):
        o_ref[s * _BLK:(s + 1) * _BLK, :] = o[:, c * s:c * (s + 1)]


def kernel(node_features, edge_features, edge_index, node_tiers,
           Wq1, Wk1, Wv1, We1, Wo1, Wq2, Wk2, Wv2, We2, Wo2,
           Wc1, bc1, Wc2, bc2):
    E, DE = edge_features.shape
    C = Wc2.shape[1]
    din = _PACK * DE
    dout = _PACK * C
    nblk = E // (_PACK * _BLK)

    eye = jnp.eye(_PACK, dtype=jnp.float32)
    w1 = jnp.kron(eye, Wc1.astype(jnp.float32))
    w2 = jnp.kron(eye, Wc2.astype(jnp.float32))
    b1 = jnp.tile(bc1.astype(jnp.float32), _PACK)[None, :]
    b2 = jnp.tile(bc2.astype(jnp.float32), _PACK)[None, :]

    in_specs = [
        pl.BlockSpec((_PACK * _BLK, DE), lambda i: (i, 0)),
        pl.BlockSpec((din, din), lambda i: (0, 0)),
        pl.BlockSpec((1, din), lambda i: (0, 0)),
        pl.BlockSpec((din, dout), lambda i: (0, 0)),
        pl.BlockSpec((1, dout), lambda i: (0, 0)),
    ]

    out = pl.pallas_call(
        _mlp_kernel,
        grid=(nblk,),
        in_specs=in_specs,
        out_specs=pl.BlockSpec((_PACK * _BLK, C), lambda i: (i, 0)),
        out_shape=jax.ShapeDtypeStruct((E, C), jnp.float32),
        compiler_params=pltpu.CompilerParams(
            dimension_semantics=("parallel",),
        ),
    )(edge_features, w1, b1, w2, b2)
    return out


# speedup vs baseline: 8.0073x; 7.9854x over previous
import jax
import jax.numpy as jnp
import numpy as np
from jax.experimental import pallas as pl
from jax.experimental.pallas import tpu as pltpu

_BLKE = 16000


def _mlp_kernel(x_ref, w1_ref, b1_ref, w2_ref, b2_ref, o_ref):
    x = x_ref[...]
    h = jnp.dot(w1_ref[...], x, preferred_element_type=jnp.float32) + b1_ref[...]
    h = 0.5 * h * (1.0 + jax.lax.erf(h * np.float32(1.0 / np.sqrt(2.0))))
    o_ref[...] = jnp.dot(w2_ref[...], h, preferred_element_type=jnp.float32) + b2_ref[...]


def kernel(node_features, edge_features, edge_index, node_tiers,
           Wq1, Wk1, Wv1, We1, Wo1, Wq2, Wk2, Wv2, We2, Wo2,
           Wc1, bc1, Wc2, bc2):
    E, DE = edge_features.shape
    C = Wc2.shape[1]
    nblk = E // _BLKE

    x_t = edge_features.T
    w1t = Wc1.astype(jnp.float32).T
    w2t = Wc2.astype(jnp.float32).T
    b1c = bc1.astype(jnp.float32)[:, None]
    b2c = bc2.astype(jnp.float32)[:, None]

    out_t = pl.pallas_call(
        _mlp_kernel,
        grid=(nblk,),
        in_specs=[
            pl.BlockSpec((DE, _BLKE), lambda i: (0, i)),
            pl.BlockSpec((DE, DE), lambda i: (0, 0)),
            pl.BlockSpec((DE, 1), lambda i: (0, 0)),
            pl.BlockSpec((C, DE), lambda i: (0, 0)),
            pl.BlockSpec((C, 1), lambda i: (0, 0)),
        ],
        out_specs=pl.BlockSpec((C, _BLKE), lambda i: (0, i)),
        out_shape=jax.ShapeDtypeStruct((C, E), jnp.float32),
        compiler_params=pltpu.CompilerParams(
            dimension_semantics=("parallel",),
        ),
    )(x_t, w1t, b1c, w2t, b2c)
    return out_t.T


# speedup vs baseline: 9.2280x; 1.1525x over previous
import jax
import jax.numpy as jnp
import numpy as np
from jax.experimental import pallas as pl
from jax.experimental.pallas import tpu as pltpu

_BLKE = 16000

_DN = (((0,), (0,)), ((), ()))


def _mlp_kernel(x_ref, w1_ref, b1_ref, w2_ref, b2_ref, o_ref):
    x = x_ref[...]
    b1 = b1_ref[...].reshape(-1, 1)
    b2 = b2_ref[...].reshape(-1, 1)
    h = jax.lax.dot_general(w1_ref[...], x, _DN,
                            preferred_element_type=jnp.float32) + b1
    h = 0.5 * h * (1.0 + jax.lax.erf(h * np.float32(1.0 / np.sqrt(2.0))))
    o_ref[...] = jax.lax.dot_general(w2_ref[...], h, _DN,
                                     preferred_element_type=jnp.float32) + b2


def kernel(node_features, edge_features, edge_index, node_tiers,
           Wq1, Wk1, Wv1, We1, Wo1, Wq2, Wk2, Wv2, We2, Wo2,
           Wc1, bc1, Wc2, bc2):
    E, DE = edge_features.shape
    C = Wc2.shape[1]
    nblk = E // _BLKE

    x_t = edge_features.T

    out_t = pl.pallas_call(
        _mlp_kernel,
        grid=(nblk,),
        in_specs=[
            pl.BlockSpec((DE, _BLKE), lambda i: (0, i)),
            pl.BlockSpec((DE, DE), lambda i: (0, 0)),
            pl.BlockSpec((DE,), lambda i: (0,)),
            pl.BlockSpec((DE, C), lambda i: (0, 0)),
            pl.BlockSpec((C,), lambda i: (0,)),
        ],
        out_specs=pl.BlockSpec((C, _BLKE), lambda i: (0, i)),
        out_shape=jax.ShapeDtypeStruct((C, E), jnp.float32),
        compiler_params=pltpu.CompilerParams(
            dimension_semantics=("parallel",),
        ),
    )(x_t, Wc1, bc1, Wc2, bc2)
    return out_t.T


# speedup vs baseline: 11.2487x; 1.2190x over previous
import jax
import jax.numpy as jnp
import numpy as np
from jax.experimental import pallas as pl
from jax.experimental.pallas import tpu as pltpu

_BLKE = 32000

_DN = (((0,), (0,)), ((), ()))


def _mlp_kernel(x_ref, w1_ref, b1_ref, w2_ref, b2_ref, o_ref):
    x = x_ref[...]
    b1 = b1_ref[...].reshape(-1, 1)
    b2 = b2_ref[...].reshape(-1, 1)
    h = jax.lax.dot_general(w1_ref[...], x, _DN,
                            preferred_element_type=jnp.float32) + b1
    h = 0.5 * h * (1.0 + jax.lax.erf(h * np.float32(1.0 / np.sqrt(2.0))))
    o_ref[...] = jax.lax.dot_general(w2_ref[...], h, _DN,
                                     preferred_element_type=jnp.float32) + b2


def kernel(node_features, edge_features, edge_index, node_tiers,
           Wq1, Wk1, Wv1, We1, Wo1, Wq2, Wk2, Wv2, We2, Wo2,
           Wc1, bc1, Wc2, bc2):
    E, DE = edge_features.shape
    C = Wc2.shape[1]
    nblk = E // _BLKE

    x_t = edge_features.T

    out_t = pl.pallas_call(
        _mlp_kernel,
        grid=(nblk,),
        in_specs=[
            pl.BlockSpec((DE, _BLKE), lambda i: (0, i)),
            pl.BlockSpec((DE, DE), lambda i: (0, 0)),
            pl.BlockSpec((DE,), lambda i: (0,)),
            pl.BlockSpec((DE, C), lambda i: (0, 0)),
            pl.BlockSpec((C,), lambda i: (0,)),
        ],
        out_specs=pl.BlockSpec((C, _BLKE), lambda i: (0, i)),
        out_shape=jax.ShapeDtypeStruct((C, E), jnp.float32),
        compiler_params=pltpu.CompilerParams(
            dimension_semantics=("parallel",),
        ),
    )(x_t, Wc1, bc1, Wc2, bc2)
    return out_t.T


# speedup vs baseline: 11.9045x; 1.0583x over previous
import jax
import jax.numpy as jnp
import numpy as np
from jax.experimental import pallas as pl
from jax.experimental.pallas import tpu as pltpu

_BLKE = 64000

_DN = (((0,), (0,)), ((), ()))


def _mlp_kernel(x_ref, w1_ref, b1_ref, w2_ref, b2_ref, o_ref):
    x = x_ref[...]
    b1 = b1_ref[...].reshape(-1, 1)
    b2 = b2_ref[...].reshape(-1, 1)
    h = jax.lax.dot_general(w1_ref[...], x, _DN,
                            preferred_element_type=jnp.float32) + b1
    h = 0.5 * h * (1.0 + jax.lax.erf(h * np.float32(1.0 / np.sqrt(2.0))))
    o_ref[...] = jax.lax.dot_general(w2_ref[...], h, _DN,
                                     preferred_element_type=jnp.float32) + b2


def kernel(node_features, edge_features, edge_index, node_tiers,
           Wq1, Wk1, Wv1, We1, Wo1, Wq2, Wk2, Wv2, We2, Wo2,
           Wc1, bc1, Wc2, bc2):
    E, DE = edge_features.shape
    C = Wc2.shape[1]
    nblk = E // _BLKE

    x_t = edge_features.T

    out_t = pl.pallas_call(
        _mlp_kernel,
        grid=(nblk,),
        in_specs=[
            pl.BlockSpec((DE, _BLKE), lambda i: (0, i)),
            pl.BlockSpec((DE, DE), lambda i: (0, 0)),
            pl.BlockSpec((DE,), lambda i: (0,)),
            pl.BlockSpec((DE, C), lambda i: (0, 0)),
            pl.BlockSpec((C,), lambda i: (0,)),
        ],
        out_specs=pl.BlockSpec((C, _BLKE), lambda i: (0, i)),
        out_shape=jax.ShapeDtypeStruct((C, E), jnp.float32),
        compiler_params=pltpu.CompilerParams(
            dimension_semantics=("parallel",),
        ),
    )(x_t, Wc1, bc1, Wc2, bc2)
    return out_t.T


# speedup vs baseline: 12.0895x; 1.0155x over previous
import jax
import jax.numpy as jnp
import numpy as np
from jax.experimental import pallas as pl
from jax.experimental.pallas import tpu as pltpu

_BLKE = 80000

_DN = (((0,), (0,)), ((), ()))


def _mlp_kernel(x_ref, w1_ref, b1_ref, w2_ref, b2_ref, o_ref):
    x = x_ref[...]
    b1 = b1_ref[...].reshape(-1, 1)
    b2 = b2_ref[...].reshape(-1, 1)
    h = jax.lax.dot_general(w1_ref[...], x, _DN,
                            preferred_element_type=jnp.float32) + b1
    h = 0.5 * h * (1.0 + jax.lax.erf(h * np.float32(1.0 / np.sqrt(2.0))))
    o_ref[...] = jax.lax.dot_general(w2_ref[...], h, _DN,
                                     preferred_element_type=jnp.float32) + b2


def kernel(node_features, edge_features, edge_index, node_tiers,
           Wq1, Wk1, Wv1, We1, Wo1, Wq2, Wk2, Wv2, We2, Wo2,
           Wc1, bc1, Wc2, bc2):
    E, DE = edge_features.shape
    C = Wc2.shape[1]
    nblk = E // _BLKE

    x_t = edge_features.T

    out_t = pl.pallas_call(
        _mlp_kernel,
        grid=(nblk,),
        in_specs=[
            pl.BlockSpec((DE, _BLKE), lambda i: (0, i)),
            pl.BlockSpec((DE, DE), lambda i: (0, 0)),
            pl.BlockSpec((DE,), lambda i: (0,)),
            pl.BlockSpec((DE, C), lambda i: (0, 0)),
            pl.BlockSpec((C,), lambda i: (0,)),
        ],
        out_specs=pl.BlockSpec((C, _BLKE), lambda i: (0, i)),
        out_shape=jax.ShapeDtypeStruct((C, E), jnp.float32),
        compiler_params=pltpu.CompilerParams(
            dimension_semantics=("parallel",),
        ),
    )(x_t, Wc1, bc1, Wc2, bc2)
    return out_t.T
